# baseline (speedup 1.0000x reference)
import functools

import jax
import jax.numpy as jnp
from jax import lax
from jax.experimental import pallas as pl
from jax.experimental.pallas import tpu as pltpu
from jax.experimental.pallas import tpu_sc as plsc

_NC, _NS, _NL = 2, 16, 16
_NW = _NC * _NS


def _sc_plane(x, emb_table):
    S, D = x.shape
    V = emb_table.shape[0]
    rows = S // _NW

    @functools.partial(
        pl.kernel,
        mesh=plsc.VectorSubcoreMesh(core_axis_name="c", subcore_axis_name="s"),
        out_type=jax.ShapeDtypeStruct((S, D), jnp.float32),
        scratch_types=[
            pltpu.VMEM((rows, D), jnp.float32),
            pltpu.VMEM((rows, D), jnp.float32),
            pltpu.VMEM((rows, D), jnp.float32),
        ],
    )
    def k(x_hbm, tab_hbm, y_hbm, xv, tv, yv):
        wid = lax.axis_index("s") * _NC + lax.axis_index("c")
        s0 = wid * rows

        @pl.when(s0 < V)
        def _():
            pltpu.sync_copy(x_hbm.at[pl.ds(s0, rows)], xv)
            pltpu.sync_copy(tab_hbm.at[pl.ds(s0, rows)], tv)
            for r in range(rows):
                for j in range(D // _NL):
                    sl = pl.ds(j * _NL, _NL)
                    yv[r, sl] = xv[r, sl] + tv[r, sl]

        @pl.when(s0 >= V)
        def _():
            nan16 = jnp.full((_NL,), jnp.nan, dtype=jnp.float32)
            for r in range(rows):
                for j in range(D // _NL):
                    yv[r, pl.ds(j * _NL, _NL)] = nan16

        pltpu.sync_copy(yv, y_hbm.at[pl.ds(s0, rows)])

    return k(x, emb_table)


def _tc_replicate(y, B):
    S, D = y.shape
    BB = 16

    def body(y_ref, o_ref):
        o_ref[...] = jnp.broadcast_to(y_ref[...][None], o_ref.shape)

    return pl.pallas_call(
        body,
        grid=(B // BB,),
        in_specs=[pl.BlockSpec((S, D), lambda i: (0, 0))],
        out_specs=pl.BlockSpec((BB, S, D), lambda i: (i, 0, 0)),
        out_shape=jax.ShapeDtypeStruct((B, S, D), y.dtype),
        compiler_params=pltpu.CompilerParams(
            dimension_semantics=("parallel",)),
    )(y)


def kernel(x, emb_table):
    B = x.shape[0]
    y = _sc_plane(x, emb_table)
    return _tc_replicate(y, B)


# speedup vs baseline: 1.0294x; 1.0294x over previous
import functools

import jax
import jax.numpy as jnp
from jax import lax
from jax.experimental import pallas as pl
from jax.experimental.pallas import tpu as pltpu
from jax.experimental.pallas import tpu_sc as plsc

_NC, _NS, _NL = 2, 16, 16
_NW = _NC * _NS


def kernel(x, emb_table):
    B, S = x.shape
    V, D = emb_table.shape
    rows = S // _NW

    @functools.partial(
        pl.kernel,
        mesh=plsc.VectorSubcoreMesh(core_axis_name="c", subcore_axis_name="s"),
        out_type=jax.ShapeDtypeStruct((B, S, D), jnp.float32),
        scratch_types=[
            pltpu.VMEM((rows, D), jnp.float32),
            pltpu.VMEM((rows, D), jnp.float32),
            pltpu.VMEM((rows, D), jnp.float32),
            pltpu.SemaphoreType.DMA,
        ],
    )
    def k(x_hbm, tab_hbm, out_hbm, xv, tv, yv, sem):
        wid = lax.axis_index("s") * _NC + lax.axis_index("c")
        s0 = wid * rows

        @pl.when(s0 < V)
        def _():
            pltpu.sync_copy(x_hbm.at[pl.ds(s0, rows)], xv)
            pltpu.sync_copy(tab_hbm.at[pl.ds(s0, rows)], tv)
            for r in range(rows):
                for j in range(D // _NL):
                    sl = pl.ds(j * _NL, _NL)
                    yv[r, sl] = xv[r, sl] + tv[r, sl]

        @pl.when(s0 >= V)
        def _():
            nan16 = jnp.full((_NL,), jnp.nan, dtype=jnp.float32)
            for r in range(rows):
                for j in range(D // _NL):
                    yv[r, pl.ds(j * _NL, _NL)] = nan16

        def fire(b, carry):
            pltpu.async_copy(yv, out_hbm.at[b, pl.ds(s0, rows)], sem)
            return carry

        lax.fori_loop(0, B, fire, 0)

        def drain(b, carry):
            pltpu.make_async_copy(yv, out_hbm.at[0, pl.ds(s0, rows)],
                                  sem).wait()
            return carry

        lax.fori_loop(0, B, drain, 0)

    return k(x, emb_table)
